# baseline (speedup 1.0000x reference)
import functools

import jax
import jax.numpy as jnp
from jax import lax
from jax.experimental import pallas as pl
from jax.experimental.pallas import tpu as pltpu
from jax.experimental.pallas import tpu_sc as plsc

B = 1024
CTX = 50
D = 32
DP = 128
V = 100000

NC = 2
NS = 16
NW = NC * NS
B_PER_W = B // NW
ROWS_PER_W = B_PER_W * CTX
N_CHUNK = 4
B_PER_CHUNK = B_PER_W // N_CHUNK
ROWS_PER_CHUNK = ROWS_PER_W // N_CHUNK

_sc_mesh = plsc.VectorSubcoreMesh(core_axis_name="c", subcore_axis_name="s")


@functools.partial(
    pl.kernel,
    out_type=jax.ShapeDtypeStruct((B * D,), jnp.float32),
    mesh=_sc_mesh,
    scratch_types=[
        pltpu.VMEM((ROWS_PER_W,), jnp.int32),
        pltpu.VMEM((2, ROWS_PER_CHUNK, DP), jnp.bfloat16),
        pltpu.VMEM((B_PER_W * D,), jnp.float32),
        pltpu.SemaphoreType.DMA,
    ],
    compiler_params=pltpu.CompilerParams(
        use_tc_tiling_on_sc=False, needs_layout_passes=False),
)
def _pool_sc(idx_hbm, table_hbm, out_hbm, idx_v, rows_v, pooled_v, sem):
    wid = lax.axis_index("s") * NC + lax.axis_index("c")
    inv = jnp.float32(1.0 / CTX)
    pltpu.sync_copy(idx_hbm.at[pl.ds(wid * ROWS_PER_W, ROWS_PER_W)], idx_v)

    def gather(ck, slot):
        return pltpu.make_async_copy(
            table_hbm.at[idx_v.at[pl.ds(ck * ROWS_PER_CHUNK, ROWS_PER_CHUNK)]],
            rows_v.at[slot], sem)

    lane2 = lax.iota(jnp.int32, 16) * 2

    gather(0, 0).start()
    for ck in range(N_CHUNK):
        slot = ck % 2
        gather(ck, slot).wait()
        if ck + 1 < N_CHUNK:
            gather(ck + 1, 1 - slot).start()

        def body_b(b, carry):
            def body_c(c, acc):
                ae, ao = acc
                r = b * CTX + c
                e, o = plsc.unpack(
                    rows_v[slot, r, pl.ds(0, D)],
                    format=plsc.PackFormat.INTERLEAVED,
                    preferred_element_type=jnp.float32,
                )
                return (ae + e, ao + o)

            ae, ao = lax.fori_loop(
                0, CTX, body_c,
                (jnp.zeros((16,), jnp.float32), jnp.zeros((16,), jnp.float32)),
            )
            off = (ck * B_PER_CHUNK + b) * D
            plsc.store_scatter(pooled_v, [lane2 + off], ae * inv)
            plsc.store_scatter(pooled_v, [lane2 + (off + 1)], ao * inv)
            return carry

        lax.fori_loop(0, B_PER_CHUNK, body_b, 0)

    pltpu.sync_copy(pooled_v, out_hbm.at[pl.ds(wid * B_PER_W * D, B_PER_W * D)])


VB2 = 8192
_N_VB2 = (V + VB2 - 1) // VB2


def _tpad_tc(xt_ref, o_ref):
    eye = jnp.eye(D, dtype=jnp.float32)
    o_ref[:, 0:D] = jax.lax.dot_general(
        xt_ref[...], eye, (((0,), (0,)), ((), ())),
        preferred_element_type=jnp.float32,
    ).astype(jnp.bfloat16)


def _transpose_pad(table_t):
    return pl.pallas_call(
        _tpad_tc,
        grid=(_N_VB2,),
        in_specs=[pl.BlockSpec((D, VB2), lambda j: (0, j))],
        out_specs=pl.BlockSpec((VB2, DP), lambda j: (j, 0)),
        out_shape=jax.ShapeDtypeStruct((V, DP), jnp.bfloat16),
    )(table_t)


VB = 4096
_N_VB = (V + VB - 1) // VB


def _matmul_tc(w_ref, x_ref, b_ref, o_ref):
    dgn = (((0,), (0,)), ((), ()))
    o_ref[...] = jax.lax.dot_general(
        w_ref[...], x_ref[...], dgn, preferred_element_type=jnp.float32
    ) + jax.lax.dot_general(
        b_ref[...], jnp.ones((1, B), jnp.float32), dgn,
        preferred_element_type=jnp.float32,
    )


@jax.jit
def kernel(inputs, emb_table, dense_W, dense_b):
    idx = inputs.reshape(-1).astype(jnp.int32)
    table_p = _transpose_pad(emb_table.T)
    pooled = _pool_sc(idx, table_p).reshape(B, D)
    logits_t = pl.pallas_call(
        _matmul_tc,
        grid=(_N_VB,),
        in_specs=[
            pl.BlockSpec((D, VB), lambda j: (0, j)),
            pl.BlockSpec((D, B), lambda j: (0, 0)),
            pl.BlockSpec((1, VB), lambda j: (0, j)),
        ],
        out_specs=pl.BlockSpec((VB, B), lambda j: (j, 0)),
        out_shape=jax.ShapeDtypeStruct((V, B), jnp.float32),
    )(dense_W, pooled.T, dense_b[None, :])
    return logits_t.T


# speedup vs baseline: 1.4709x; 1.4709x over previous
import functools

import jax
import jax.numpy as jnp
from jax import lax
from jax.experimental import pallas as pl
from jax.experimental.pallas import tpu as pltpu
from jax.experimental.pallas import tpu_sc as plsc

B = 1024
CTX = 50
D = 32
DP = 128
V = 100000

NC = 2
NS = 16
NW = NC * NS
B_PER_W = B // NW
ROWS_PER_W = B_PER_W * CTX
N_CHUNK = 4
B_PER_CHUNK = B_PER_W // N_CHUNK
ROWS_PER_CHUNK = ROWS_PER_W // N_CHUNK

_sc_mesh = plsc.VectorSubcoreMesh(core_axis_name="c", subcore_axis_name="s")


@functools.partial(
    pl.kernel,
    out_type=jax.ShapeDtypeStruct((B, D), jnp.float32),
    mesh=_sc_mesh,
    scratch_types=[
        pltpu.VMEM((ROWS_PER_W,), jnp.int32),
        pltpu.VMEM((2, ROWS_PER_CHUNK, D), jnp.float32),
        pltpu.VMEM((B_PER_W, D), jnp.float32),
        pltpu.SemaphoreType.DMA,
    ],
    compiler_params=pltpu.CompilerParams(use_tc_tiling_on_sc=False),
)
def _pool_sc(idx_hbm, table_hbm, out_hbm, idx_v, rows_v, pooled_v, sem):
    wid = lax.axis_index("s") * NC + lax.axis_index("c")
    inv = jnp.float32(1.0 / CTX)
    pltpu.sync_copy(idx_hbm.at[pl.ds(wid * ROWS_PER_W, ROWS_PER_W)], idx_v)

    def gather(ck, slot):
        return pltpu.make_async_copy(
            table_hbm.at[idx_v.at[pl.ds(ck * ROWS_PER_CHUNK, ROWS_PER_CHUNK)]],
            rows_v.at[slot], sem)

    gather(0, 0).start()
    for ck in range(N_CHUNK):
        slot = ck % 2
        gather(ck, slot).wait()
        if ck + 1 < N_CHUNK:
            gather(ck + 1, 1 - slot).start()

        def body_b(b, carry):
            def body_c(c, acc):
                a0, a1 = acc
                r = b * CTX + c
                a0 = a0 + rows_v[slot, r, pl.ds(0, 16)]
                a1 = a1 + rows_v[slot, r, pl.ds(16, 16)]
                return (a0, a1)

            a0, a1 = lax.fori_loop(
                0, CTX, body_c,
                (jnp.zeros((16,), jnp.float32), jnp.zeros((16,), jnp.float32)),
            )
            bb = ck * B_PER_CHUNK + b
            pooled_v[bb, pl.ds(0, 16)] = a0 * inv
            pooled_v[bb, pl.ds(16, 16)] = a1 * inv
            return carry

        lax.fori_loop(0, B_PER_CHUNK, body_b, 0)

    pltpu.sync_copy(pooled_v, out_hbm.at[pl.ds(wid * B_PER_W, B_PER_W)])


G4 = 4
G = 25600
VP = G * G4
BQ = 3200
_N_BQ = G // BQ


def _tpack_tc(x0_ref, x1_ref, x2_ref, x3_ref, o_ref):
    eye = jnp.eye(D, dtype=jnp.float32)
    dgn = (((0,), (0,)), ((), ()))
    parts = [
        jax.lax.dot_general(x[...], eye, dgn,
                            preferred_element_type=jnp.float32)
        for x in (x0_ref, x1_ref, x2_ref, x3_ref)
    ]
    o_ref[...] = jnp.concatenate(parts, axis=1)


def _transpose_pack(table_t):
    def spec(s):
        return pl.BlockSpec((D, BQ), lambda j, s=s: (0, s * _N_BQ + j))

    return pl.pallas_call(
        _tpack_tc,
        grid=(_N_BQ,),
        in_specs=[spec(0), spec(1), spec(2), spec(3)],
        out_specs=pl.BlockSpec((BQ, G4 * D), lambda j: (j, 0)),
        out_shape=jax.ShapeDtypeStruct((G, G4 * D), jnp.float32),
    )(table_t, table_t, table_t, table_t)


VB = 4096
_N_VB = (V + VB - 1) // VB


def _matmul_tc(w_ref, x_ref, b_ref, o_ref):
    dgn = (((0,), (0,)), ((), ()))
    o_ref[...] = jax.lax.dot_general(
        w_ref[...], x_ref[...], dgn, preferred_element_type=jnp.float32
    ) + jax.lax.dot_general(
        b_ref[...], jnp.ones((1, B), jnp.float32), dgn,
        preferred_element_type=jnp.float32,
    )


@jax.jit
def kernel(inputs, emb_table, dense_W, dense_b):
    r = inputs.reshape(-1).astype(jnp.int32)
    idx = (r % G) * G4 + r // G
    table_lin = _transpose_pack(emb_table.T).reshape(VP, D)
    pooled = _pool_sc(idx, table_lin)
    logits_t = pl.pallas_call(
        _matmul_tc,
        grid=(_N_VB,),
        in_specs=[
            pl.BlockSpec((D, VB), lambda j: (0, j)),
            pl.BlockSpec((D, B), lambda j: (0, 0)),
            pl.BlockSpec((1, VB), lambda j: (0, j)),
        ],
        out_specs=pl.BlockSpec((VB, B), lambda j: (j, 0)),
        out_shape=jax.ShapeDtypeStruct((V, B), jnp.float32),
    )(dense_W, pooled.T, dense_b[None, :])
    return logits_t.T
